# baseline (speedup 1.0000x reference)
import functools

import jax
import jax.numpy as jnp
from jax import lax
from jax.experimental import pallas as pl
from jax.experimental.pallas import tpu as pltpu
from jax.experimental.pallas import tpu_sc as plsc

N = 65536
T = 2097152

NC = 2
NS = 16
L = 16
NW = NC * NS

PER_W = T // NW
CHUNK = 2048
N_CHUNKS = PER_W // CHUNK
GROUPS = CHUNK // L
DMA_SLICE = 128
N_DMA = CHUNK // DMA_SLICE

_INV_PI = 0.3183098861837907
_PI_HI = 3.140625
_PI_LO = 9.676535897932795e-4
_S1, _S2, _S3 = -1.6666667e-1, 8.3333333e-3, -1.9841270e-4
_C1, _C2, _C3, _C4 = -0.5, 4.1666667e-2, -1.3888889e-3, 2.4801587e-5
_MAGIC = 12582912.0


def _sincos(x):
    kf = (x * _INV_PI + _MAGIC) - _MAGIC
    ki = kf.astype(jnp.int32)
    r = x - kf * _PI_HI
    r = r - kf * _PI_LO
    r2 = r * r
    ps = ((_S3 * r2 + _S2) * r2 + _S1) * r2 + 1.0
    pc = (((_C4 * r2 + _C3) * r2 + _C2) * r2 + _C1) * r2 + 1.0
    sinr = r * ps
    sgn = jnp.where((ki & 1) != 0, -1.0, 1.0)
    return sinr, pc, sgn


R_TILE = N // NS
R_STAGE = R_TILE + 8
R_SUB = 2048


def _body(grad_hbm, amp_hbm, ph_hbm, t_hbm,
          packed_hbm, gx_hbm, gy_hbm, gz_hbm, re_hbm, im_hbm,
          grad_s, amp_s, ph_s,
          t_A, idx_A, frac_A, rows_A, t_B, idx_B, frac_B, rows_B,
          gx_v, gy_v, gz_v, re_v, im_v, sem_A, sem_B):
    s_ax = lax.axis_index("s")
    cc = lax.axis_index("c")
    wid = s_ax * NC + cc
    w_base = wid * PER_W
    bufs = {0: (t_A, idx_A, frac_A, rows_A, sem_A),
            1: (t_B, idx_B, frac_B, rows_B, sem_B)}

    rs = s_ax * R_TILE
    d = jnp.maximum(rs + R_STAGE - N, 0)
    start = pl.multiple_of(rs - d, 8)
    pltpu.sync_copy(grad_hbm.at[pl.ds(start, R_STAGE)], grad_s)
    pltpu.sync_copy(amp_hbm.at[pl.ds(start, R_STAGE)], amp_s)
    pltpu.sync_copy(ph_hbm.at[pl.ds(start, R_STAGE)], ph_s)
    prow0 = cc * N + rs
    for sb in range(R_TILE // R_SUB):
        rows_st = (rows_A, rows_B)[sb % 2]

        @plsc.parallel_loop(0, R_SUB, L, unroll=2)
        def packloop(i):
            lr = lax.iota(jnp.int32, L) + (i + sb * R_SUB + d)
            ri = lax.iota(jnp.int32, L) + i

            def cfull(v):
                return jnp.full((L,), v, jnp.int32)

            for c in range(3):
                cur = plsc.load_gather(grad_s, [lr, cfull(c)])
                nxt = plsc.load_gather(grad_s, [lr + 1, cfull(c)])
                plsc.store_scatter(rows_st, [ri, cfull(c)], cur)
                plsc.store_scatter(rows_st, [ri, cfull(c + 5)], nxt)
            plsc.store_scatter(rows_st, [ri, cfull(3)],
                               plsc.load_gather(amp_s, [lr]))
            plsc.store_scatter(rows_st, [ri, cfull(8)],
                               plsc.load_gather(amp_s, [lr + 1]))
            plsc.store_scatter(rows_st, [ri, cfull(4)],
                               plsc.load_gather(ph_s, [lr]))
            plsc.store_scatter(rows_st, [ri, cfull(9)],
                               plsc.load_gather(ph_s, [lr + 1]))

        pltpu.sync_copy(rows_st,
                        packed_hbm.at[pl.ds(prow0 + sb * R_SUB, R_SUB)])
    plsc.subcore_barrier()

    def prep_fire(p, base):
        t_v, idx_v, frac_v, rows_v, sem = bufs[p]
        pltpu.sync_copy(t_hbm.at[pl.ds(base, CHUNK)], t_v)

        @plsc.parallel_loop(0, CHUNK, L, unroll=4)
        def pass1(i):
            tv = t_v[pl.ds(i, L)]
            ti = tv.astype(jnp.int32)
            ti = jnp.minimum(jnp.maximum(ti, 0), N - 2)
            frac_v[pl.ds(i, L)] = tv - ti.astype(jnp.float32)
            idx_v[pl.ds(i, L)] = ti + cc * N

        for d in range(N_DMA):
            pltpu.async_copy(
                packed_hbm.at[idx_v.at[pl.ds(d * DMA_SLICE, DMA_SLICE)]],
                rows_v.at[pl.ds(d * DMA_SLICE, DMA_SLICE)],
                sem)

    def finish(p, base):
        t_v, idx_v, frac_v, rows_v, sem = bufs[p]
        for d in range(N_DMA):
            pltpu.make_async_copy(
                packed_hbm.at[idx_v.at[pl.ds(d * DMA_SLICE, DMA_SLICE)]],
                rows_v.at[pl.ds(d * DMA_SLICE, DMA_SLICE)],
                sem).wait()

        @plsc.parallel_loop(0, CHUNK, L, unroll=4)
        def pass2(i):
            sl = pl.ds(i, L)
            fr = frac_v[sl]
            ri = lax.iota(jnp.int32, L) + i

            def col(cc):
                ci = jnp.full((L,), cc, jnp.int32)
                return plsc.load_gather(rows_v, [ri, ci])

            def lerp(cc):
                a = col(cc)
                b = col(cc + 5)
                return a + fr * (b - a)

            gx_v[sl] = lerp(0)
            gy_v[sl] = lerp(1)
            gz_v[sl] = lerp(2)
            amp = lerp(3)
            ph = lerp(4)
            sinr, cosr, sgn = _sincos(ph)
            amps = amp * sgn
            re_v[sl] = amps * cosr
            im_v[sl] = amps * sinr

        pltpu.sync_copy(gx_v, gx_hbm.at[pl.ds(base, CHUNK)])
        pltpu.sync_copy(gy_v, gy_hbm.at[pl.ds(base, CHUNK)])
        pltpu.sync_copy(gz_v, gz_hbm.at[pl.ds(base, CHUNK)])
        pltpu.sync_copy(re_v, re_hbm.at[pl.ds(base, CHUNK)])
        pltpu.sync_copy(im_v, im_hbm.at[pl.ds(base, CHUNK)])

    prep_fire(0, w_base)

    def pair_body(k, carry):
        base_a = w_base + 2 * k * CHUNK
        prep_fire(1, base_a + CHUNK)
        finish(0, base_a)

        @pl.when(k < N_CHUNKS // 2 - 1)
        def _():
            prep_fire(0, base_a + 2 * CHUNK)

        finish(1, base_a + CHUNK)
        return carry

    lax.fori_loop(0, N_CHUNKS // 2, pair_body, 0)


_mesh = plsc.VectorSubcoreMesh(
    core_axis_name="c", subcore_axis_name="s", num_cores=NC, num_subcores=NS)

_sc_interp = functools.partial(
    pl.kernel,
    out_type=[jax.ShapeDtypeStruct((NC * N, 16), jnp.float32)]
    + [jax.ShapeDtypeStruct((T,), jnp.float32)] * 5,
    mesh=_mesh,
    compiler_params=pltpu.CompilerParams(
        use_tc_tiling_on_sc=False, needs_layout_passes=False),
    scratch_types=[
        pltpu.VMEM((R_STAGE, 3), jnp.float32),
        pltpu.VMEM((R_STAGE,), jnp.float32),
        pltpu.VMEM((R_STAGE,), jnp.float32),
        pltpu.VMEM((CHUNK,), jnp.float32),
        pltpu.VMEM((CHUNK,), jnp.int32),
        pltpu.VMEM((CHUNK,), jnp.float32),
        pltpu.VMEM((CHUNK, 16), jnp.float32),
        pltpu.VMEM((CHUNK,), jnp.float32),
        pltpu.VMEM((CHUNK,), jnp.int32),
        pltpu.VMEM((CHUNK,), jnp.float32),
        pltpu.VMEM((CHUNK, 16), jnp.float32),
        pltpu.VMEM((CHUNK,), jnp.float32),
        pltpu.VMEM((CHUNK,), jnp.float32),
        pltpu.VMEM((CHUNK,), jnp.float32),
        pltpu.VMEM((CHUNK,), jnp.float32),
        pltpu.VMEM((CHUNK,), jnp.float32),
        pltpu.SemaphoreType.DMA,
        pltpu.SemaphoreType.DMA,
    ],
)(_body)


_BT = 65536


def _stack_body(gx_ref, gy_ref, gz_ref, out_ref):
    out_ref[0, :] = gx_ref[:]
    out_ref[1, :] = gy_ref[:]
    out_ref[2, :] = gz_ref[:]


_stack3 = pl.pallas_call(
    _stack_body,
    grid=(T // _BT,),
    in_specs=[pl.BlockSpec((_BT,), lambda j: (j,))] * 3,
    out_specs=pl.BlockSpec((3, _BT), lambda j: (0, j)),
    out_shape=jax.ShapeDtypeStruct((3, T), jnp.float32),
)


def kernel(time_points, gradients, rf_amplitude, rf_phase, adc_mask, t):
    _, gx, gy, gz, re, im = _sc_interp(gradients, rf_amplitude, rf_phase, t)
    g = _stack3(gx, gy, gz)
    rf = lax.complex(re, im)
    return (g, rf)


# speedup vs baseline: 1.1576x; 1.1576x over previous
import functools

import jax
import jax.numpy as jnp
from jax import lax
from jax.experimental import pallas as pl
from jax.experimental.pallas import tpu as pltpu
from jax.experimental.pallas import tpu_sc as plsc

N = 65536
T = 2097152

NC = 2
NS = 16
L = 16
NW = NC * NS

PER_W = T // NW
CHUNK = 2048
N_CHUNKS = PER_W // CHUNK
GROUPS = CHUNK // L
DMA_SLICE = 128
N_DMA = CHUNK // DMA_SLICE

_INV_PI = 0.3183098861837907
_PI_HI = 3.140625
_PI_LO = 9.676535897932795e-4
_S1, _S2, _S3 = -1.6666667e-1, 8.3333333e-3, -1.9841270e-4
_C1, _C2, _C3, _C4 = -0.5, 4.1666667e-2, -1.3888889e-3, 2.4801587e-5
_MAGIC = 12582912.0


def _sincos(x):
    kf = (x * _INV_PI + _MAGIC) - _MAGIC
    ki = kf.astype(jnp.int32)
    r = x - kf * _PI_HI
    r = r - kf * _PI_LO
    r2 = r * r
    ps = ((_S3 * r2 + _S2) * r2 + _S1) * r2 + 1.0
    pc = (((_C4 * r2 + _C3) * r2 + _C2) * r2 + _C1) * r2 + 1.0
    sinr = r * ps
    sgn = jnp.where((ki & 1) != 0, -1.0, 1.0)
    return sinr, pc, sgn


R_TILE = N // NS
R_STAGE = R_TILE + 8
R_SUB = 2048


def _body(gxt_hbm, gyt_hbm, gzt_hbm, amp_hbm, ph_hbm, t_hbm,
          packed_hbm, gx_hbm, gy_hbm, gz_hbm, re_hbm, im_hbm,
          gx_s, gy_s, gz_s, amp_s, ph_s,
          t_A, idx_A, frac_A, rows_A, t_B, idx_B, frac_B, rows_B,
          gx_v, gy_v, gz_v, re_v, im_v, sem_A, sem_B):
    s_ax = lax.axis_index("s")
    cc = lax.axis_index("c")
    wid = s_ax * NC + cc
    w_base = wid * PER_W
    bufs = {0: (t_A, idx_A, frac_A, rows_A, sem_A),
            1: (t_B, idx_B, frac_B, rows_B, sem_B)}

    rs = s_ax * R_TILE
    d = jnp.maximum(rs + R_STAGE - N, 0)
    start = pl.multiple_of(rs - d, 8)
    pltpu.sync_copy(gxt_hbm.at[pl.ds(start, R_STAGE)], gx_s)
    pltpu.sync_copy(gyt_hbm.at[pl.ds(start, R_STAGE)], gy_s)
    pltpu.sync_copy(gzt_hbm.at[pl.ds(start, R_STAGE)], gz_s)
    pltpu.sync_copy(amp_hbm.at[pl.ds(start, R_STAGE)], amp_s)
    pltpu.sync_copy(ph_hbm.at[pl.ds(start, R_STAGE)], ph_s)
    prow0 = cc * N + rs
    for sb in range(R_TILE // R_SUB):
        rows_st = (rows_A, rows_B)[sb % 2]

        @plsc.parallel_loop(0, R_SUB, L, unroll=2)
        def packloop(i):
            lr = lax.iota(jnp.int32, L) + (i + sb * R_SUB + d)
            ri = lax.iota(jnp.int32, L) + i

            def cfull(v):
                return jnp.full((L,), v, jnp.int32)

            for c, src in enumerate((gx_s, gy_s, gz_s, amp_s, ph_s)):
                plsc.store_scatter(rows_st, [ri, cfull(c)],
                                   plsc.load_gather(src, [lr]))
                plsc.store_scatter(rows_st, [ri, cfull(c + 5)],
                                   plsc.load_gather(src, [lr + 1]))

        pltpu.sync_copy(rows_st,
                        packed_hbm.at[pl.ds(prow0 + sb * R_SUB, R_SUB)])
    plsc.subcore_barrier()

    def prep_fire(p, base):
        t_v, idx_v, frac_v, rows_v, sem = bufs[p]
        pltpu.sync_copy(t_hbm.at[pl.ds(base, CHUNK)], t_v)

        @plsc.parallel_loop(0, CHUNK, L, unroll=4)
        def pass1(i):
            tv = t_v[pl.ds(i, L)]
            ti = tv.astype(jnp.int32)
            ti = jnp.minimum(jnp.maximum(ti, 0), N - 2)
            frac_v[pl.ds(i, L)] = tv - ti.astype(jnp.float32)
            idx_v[pl.ds(i, L)] = ti + cc * N

        for d in range(N_DMA):
            pltpu.async_copy(
                packed_hbm.at[idx_v.at[pl.ds(d * DMA_SLICE, DMA_SLICE)]],
                rows_v.at[pl.ds(d * DMA_SLICE, DMA_SLICE)],
                sem)

    def finish(p, base):
        t_v, idx_v, frac_v, rows_v, sem = bufs[p]
        for d in range(N_DMA):
            pltpu.make_async_copy(
                packed_hbm.at[idx_v.at[pl.ds(d * DMA_SLICE, DMA_SLICE)]],
                rows_v.at[pl.ds(d * DMA_SLICE, DMA_SLICE)],
                sem).wait()

        @plsc.parallel_loop(0, CHUNK, L, unroll=4)
        def pass2(i):
            sl = pl.ds(i, L)
            fr = frac_v[sl]
            ri = lax.iota(jnp.int32, L) + i

            def col(cc):
                ci = jnp.full((L,), cc, jnp.int32)
                return plsc.load_gather(rows_v, [ri, ci])

            def lerp(cc):
                a = col(cc)
                b = col(cc + 5)
                return a + fr * (b - a)

            gx_v[sl] = lerp(0)
            gy_v[sl] = lerp(1)
            gz_v[sl] = lerp(2)
            amp = lerp(3)
            ph = lerp(4)
            sinr, cosr, sgn = _sincos(ph)
            amps = amp * sgn
            re_v[sl] = amps * cosr
            im_v[sl] = amps * sinr

        pltpu.sync_copy(gx_v, gx_hbm.at[pl.ds(base, CHUNK)])
        pltpu.sync_copy(gy_v, gy_hbm.at[pl.ds(base, CHUNK)])
        pltpu.sync_copy(gz_v, gz_hbm.at[pl.ds(base, CHUNK)])
        pltpu.sync_copy(re_v, re_hbm.at[pl.ds(base, CHUNK)])
        pltpu.sync_copy(im_v, im_hbm.at[pl.ds(base, CHUNK)])

    prep_fire(0, w_base)

    def pair_body(k, carry):
        base_a = w_base + 2 * k * CHUNK
        prep_fire(1, base_a + CHUNK)
        finish(0, base_a)

        @pl.when(k < N_CHUNKS // 2 - 1)
        def _():
            prep_fire(0, base_a + 2 * CHUNK)

        finish(1, base_a + CHUNK)
        return carry

    lax.fori_loop(0, N_CHUNKS // 2, pair_body, 0)


_mesh = plsc.VectorSubcoreMesh(
    core_axis_name="c", subcore_axis_name="s", num_cores=NC, num_subcores=NS)

_sc_interp = functools.partial(
    pl.kernel,
    out_type=[jax.ShapeDtypeStruct((NC * N, 16), jnp.float32)]
    + [jax.ShapeDtypeStruct((T,), jnp.float32)] * 5,
    mesh=_mesh,
    compiler_params=pltpu.CompilerParams(
        use_tc_tiling_on_sc=False, needs_layout_passes=False),
    scratch_types=[
        pltpu.VMEM((R_STAGE,), jnp.float32),
        pltpu.VMEM((R_STAGE,), jnp.float32),
        pltpu.VMEM((R_STAGE,), jnp.float32),
        pltpu.VMEM((R_STAGE,), jnp.float32),
        pltpu.VMEM((R_STAGE,), jnp.float32),
        pltpu.VMEM((CHUNK,), jnp.float32),
        pltpu.VMEM((CHUNK,), jnp.int32),
        pltpu.VMEM((CHUNK,), jnp.float32),
        pltpu.VMEM((CHUNK, 16), jnp.float32),
        pltpu.VMEM((CHUNK,), jnp.float32),
        pltpu.VMEM((CHUNK,), jnp.int32),
        pltpu.VMEM((CHUNK,), jnp.float32),
        pltpu.VMEM((CHUNK, 16), jnp.float32),
        pltpu.VMEM((CHUNK,), jnp.float32),
        pltpu.VMEM((CHUNK,), jnp.float32),
        pltpu.VMEM((CHUNK,), jnp.float32),
        pltpu.VMEM((CHUNK,), jnp.float32),
        pltpu.VMEM((CHUNK,), jnp.float32),
        pltpu.SemaphoreType.DMA,
        pltpu.SemaphoreType.DMA,
    ],
)(_body)


_BT = 65536


def _stack_body(gx_ref, gy_ref, gz_ref, out_ref):
    out_ref[0, :] = gx_ref[:]
    out_ref[1, :] = gy_ref[:]
    out_ref[2, :] = gz_ref[:]


_stack3 = pl.pallas_call(
    _stack_body,
    grid=(T // _BT,),
    in_specs=[pl.BlockSpec((_BT,), lambda j: (j,))] * 3,
    out_specs=pl.BlockSpec((3, _BT), lambda j: (0, j)),
    out_shape=jax.ShapeDtypeStruct((3, T), jnp.float32),
)


def kernel(time_points, gradients, rf_amplitude, rf_phase, adc_mask, t):
    _, gx, gy, gz, re, im = _sc_interp(
        gradients[:, 0], gradients[:, 1], gradients[:, 2],
        rf_amplitude, rf_phase, t)
    g = _stack3(gx, gy, gz)
    rf = lax.complex(re, im)
    return (g, rf)


# speedup vs baseline: 1.1576x; 1.0001x over previous
import functools

import jax
import jax.numpy as jnp
from jax import lax
from jax.experimental import pallas as pl
from jax.experimental.pallas import tpu as pltpu
from jax.experimental.pallas import tpu_sc as plsc

N = 65536
T = 2097152

NC = 2
NS = 16
L = 16
NW = NC * NS

PER_W = T // NW
CHUNK = 2048
N_CHUNKS = PER_W // CHUNK
GROUPS = CHUNK // L
DMA_SLICE = 128
N_DMA = CHUNK // DMA_SLICE

_INV_PI = 0.3183098861837907
_PI_HI = 3.140625
_PI_LO = 9.676535897932795e-4
_S1, _S2, _S3 = -1.6666667e-1, 8.3333333e-3, -1.9841270e-4
_C1, _C2, _C3, _C4 = -0.5, 4.1666667e-2, -1.3888889e-3, 2.4801587e-5
_MAGIC = 12582912.0


def _sincos(x):
    kf = (x * _INV_PI + _MAGIC) - _MAGIC
    ki = kf.astype(jnp.int32)
    r = x - kf * _PI_HI
    r = r - kf * _PI_LO
    r2 = r * r
    ps = ((_S3 * r2 + _S2) * r2 + _S1) * r2 + 1.0
    pc = (((_C4 * r2 + _C3) * r2 + _C2) * r2 + _C1) * r2 + 1.0
    sinr = r * ps
    sgn = jnp.where((ki & 1) != 0, -1.0, 1.0)
    return sinr, pc, sgn


R_TILE = N // NS
R_STAGE = R_TILE + 8
R_SUB = 2048


def _body(gxt_hbm, gyt_hbm, gzt_hbm, amp_hbm, ph_hbm, t_hbm,
          packed_hbm, gx_hbm, gy_hbm, gz_hbm, re_hbm, im_hbm,
          gx_s, gy_s, gz_s, amp_s, ph_s,
          t_A, idx_A, frac_A, rows_A, t_B, idx_B, frac_B, rows_B,
          gx_v, gy_v, gz_v, re_v, im_v, sem_A, sem_B):
    s_ax = lax.axis_index("s")
    cc = lax.axis_index("c")
    wid = s_ax * NC + cc
    w_base = wid * PER_W
    bufs = {0: (t_A, idx_A, frac_A, rows_A, sem_A),
            1: (t_B, idx_B, frac_B, rows_B, sem_B)}

    rs = s_ax * R_TILE
    d = jnp.maximum(rs + R_STAGE - N, 0)
    start = pl.multiple_of(rs - d, 8)
    pltpu.sync_copy(gxt_hbm.at[pl.ds(start, R_STAGE)], gx_s)
    pltpu.sync_copy(gyt_hbm.at[pl.ds(start, R_STAGE)], gy_s)
    pltpu.sync_copy(gzt_hbm.at[pl.ds(start, R_STAGE)], gz_s)
    pltpu.sync_copy(amp_hbm.at[pl.ds(start, R_STAGE)], amp_s)
    pltpu.sync_copy(ph_hbm.at[pl.ds(start, R_STAGE)], ph_s)
    prow0 = cc * N + rs
    for sb in range(R_TILE // R_SUB):
        rows_st = (rows_A, rows_B)[sb % 2]

        @plsc.parallel_loop(0, R_SUB, L, unroll=2)
        def packloop(i):
            lr = lax.iota(jnp.int32, L) + (i + sb * R_SUB + d)
            ri = lax.iota(jnp.int32, L) + i

            def cfull(v):
                return jnp.full((L,), v, jnp.int32)

            for c, src in enumerate((gx_s, gy_s, gz_s, amp_s, ph_s)):
                plsc.store_scatter(rows_st, [ri, cfull(c)],
                                   plsc.load_gather(src, [lr]))
                plsc.store_scatter(rows_st, [ri, cfull(c + 5)],
                                   plsc.load_gather(src, [lr + 1]))

        pltpu.sync_copy(rows_st,
                        packed_hbm.at[pl.ds(prow0 + sb * R_SUB, R_SUB)])
    plsc.subcore_barrier()

    def prep_fire(p, base):
        t_v, idx_v, frac_v, rows_v, sem = bufs[p]
        pltpu.sync_copy(t_hbm.at[pl.ds(base, CHUNK)], t_v)

        @plsc.parallel_loop(0, CHUNK, L, unroll=4)
        def pass1(i):
            tv = t_v[pl.ds(i, L)]
            ti = tv.astype(jnp.int32)
            ti = jnp.minimum(jnp.maximum(ti, 0), N - 2)
            frac_v[pl.ds(i, L)] = tv - ti.astype(jnp.float32)
            idx_v[pl.ds(i, L)] = ti + cc * N

        for d in range(N_DMA):
            pltpu.async_copy(
                packed_hbm.at[idx_v.at[pl.ds(d * DMA_SLICE, DMA_SLICE)]],
                rows_v.at[pl.ds(d * DMA_SLICE, DMA_SLICE)],
                sem)

    def finish(p, base):
        t_v, idx_v, frac_v, rows_v, sem = bufs[p]
        for d in range(N_DMA):
            pltpu.make_async_copy(
                packed_hbm.at[idx_v.at[pl.ds(d * DMA_SLICE, DMA_SLICE)]],
                rows_v.at[pl.ds(d * DMA_SLICE, DMA_SLICE)],
                sem).wait()

        @plsc.parallel_loop(0, CHUNK, L, unroll=4)
        def pass2(i):
            sl = pl.ds(i, L)
            fr = frac_v[sl]
            ri = lax.iota(jnp.int32, L) + i

            def col(cc):
                ci = jnp.full((L,), cc, jnp.int32)
                return plsc.load_gather(rows_v, [ri, ci])

            def lerp(cc):
                a = col(cc)
                b = col(cc + 5)
                return a + fr * (b - a)

            gx_v[sl] = lerp(0)
            gy_v[sl] = lerp(1)
            gz_v[sl] = lerp(2)
            amp = lerp(3)
            ph = lerp(4)
            sinr, cosr, sgn = _sincos(ph)
            amps = amp * sgn
            re_v[sl] = amps * cosr
            im_v[sl] = amps * sinr

        pltpu.sync_copy(gx_v, gx_hbm.at[pl.ds(base, CHUNK)])
        pltpu.sync_copy(gy_v, gy_hbm.at[pl.ds(base, CHUNK)])
        pltpu.sync_copy(gz_v, gz_hbm.at[pl.ds(base, CHUNK)])
        pltpu.sync_copy(re_v, re_hbm.at[pl.ds(base, CHUNK)])
        pltpu.sync_copy(im_v, im_hbm.at[pl.ds(base, CHUNK)])

    prep_fire(0, w_base)

    def pair_body(k, carry):
        base_a = w_base + 2 * k * CHUNK
        prep_fire(1, base_a + CHUNK)
        finish(0, base_a)

        @pl.when(k < N_CHUNKS // 2 - 1)
        def _():
            prep_fire(0, base_a + 2 * CHUNK)

        finish(1, base_a + CHUNK)
        return carry

    lax.fori_loop(0, N_CHUNKS // 2, pair_body, 0)


_mesh = plsc.VectorSubcoreMesh(
    core_axis_name="c", subcore_axis_name="s", num_cores=NC, num_subcores=NS)

_sc_interp = functools.partial(
    pl.kernel,
    out_type=[jax.ShapeDtypeStruct((NC * N, 16), jnp.float32)]
    + [jax.ShapeDtypeStruct((T,), jnp.float32)] * 5,
    mesh=_mesh,
    compiler_params=pltpu.CompilerParams(
        use_tc_tiling_on_sc=False, needs_layout_passes=False),
    scratch_types=[
        pltpu.VMEM((R_STAGE,), jnp.float32),
        pltpu.VMEM((R_STAGE,), jnp.float32),
        pltpu.VMEM((R_STAGE,), jnp.float32),
        pltpu.VMEM((R_STAGE,), jnp.float32),
        pltpu.VMEM((R_STAGE,), jnp.float32),
        pltpu.VMEM((CHUNK,), jnp.float32),
        pltpu.VMEM((CHUNK,), jnp.int32),
        pltpu.VMEM((CHUNK,), jnp.float32),
        pltpu.VMEM((CHUNK, 16), jnp.float32),
        pltpu.VMEM((CHUNK,), jnp.float32),
        pltpu.VMEM((CHUNK,), jnp.int32),
        pltpu.VMEM((CHUNK,), jnp.float32),
        pltpu.VMEM((CHUNK, 16), jnp.float32),
        pltpu.VMEM((CHUNK,), jnp.float32),
        pltpu.VMEM((CHUNK,), jnp.float32),
        pltpu.VMEM((CHUNK,), jnp.float32),
        pltpu.VMEM((CHUNK,), jnp.float32),
        pltpu.VMEM((CHUNK,), jnp.float32),
        pltpu.SemaphoreType.DMA,
        pltpu.SemaphoreType.DMA,
    ],
)(_body)


_BT = 65536


def _stack_body(gx_ref, gy_ref, gz_ref, out_ref):
    out_ref[0, :] = gx_ref[:]
    out_ref[1, :] = gy_ref[:]
    out_ref[2, :] = gz_ref[:]


_stack3 = pl.pallas_call(
    _stack_body,
    grid=(T // _BT,),
    in_specs=[pl.BlockSpec((_BT,), lambda j: (j,))] * 3,
    out_specs=pl.BlockSpec((3, _BT), lambda j: (0, j)),
    out_shape=jax.ShapeDtypeStruct((3, T), jnp.float32),
)


def kernel(time_points, gradients, rf_amplitude, rf_phase, adc_mask, t):
    _, gx, gy, gz, re, im = _sc_interp(
        gradients[:, 0], gradients[:, 1], gradients[:, 2],
        rf_amplitude, rf_phase, t)
    g = _stack3(gx, gy, gz)
    rf = lax.complex(re.reshape(T // 512, 512),
                     im.reshape(T // 512, 512)).reshape(T)
    return (g, rf)
